# baseline (speedup 1.0000x reference)
import functools

import jax
import jax.numpy as jnp
from jax import lax
from jax.experimental import pallas as pl
from jax.experimental.pallas import tpu as pltpu
from jax.experimental.pallas import tpu_sc as plsc

N = 10000
D = 128
E = 320000
NC = 2
NS = 16
NW = NC * NS
EPW = E // NW
CHUNK = 80
NCHUNK = EPW // CHUNK
RPT = 624


_GATHER_DNUMS = lax.GatherDimensionNumbers(
    offset_dims=(), collapsed_slice_dims=(0,), start_index_map=(0,))


def _bcast_lane(v16, i):
    idx = jnp.full((16, 1), i, jnp.int32)
    return lax.gather(v16, idx, _GATHER_DNUMS, (1,),
                      mode=lax.GatherScatterMode.PROMISE_IN_BOUNDS)


def _spmm_body(m_hbm, cols_hbm, rows_hbm, vals_hbm, out_hbm,
               colv0, rowv0, valv0, gbuf0,
               colv1, rowv1, valv1, gbuf1,
               colv2, rowv2, valv2, gbuf2,
               srow0, srow1, srow2,
               acc,
               semi0, semg0, sems0,
               semi1, semg1, sems1,
               semi2, semg2, sems2):
    c = lax.axis_index("c")
    s = lax.axis_index("s")
    w = s * NC + c

    colv = (colv0, colv1, colv2)
    rowv = (rowv0, rowv1, rowv2)
    valv = (valv0, valv1, valv2)
    gbuf = (gbuf0, gbuf1, gbuf2)
    semi = (semi0, semi1, semi2)
    semg = (semg0, semg1, semg2)
    sems = (sems0, sems1, sems2)
    srow = (srow0, srow1, srow2)

    def _zrow(i, carry):
        for j in range(D // 16):
            gbuf0[i, pl.ds(j * 16, 16)] = jnp.zeros((16,), jnp.float32)
        return carry
    lax.fori_loop(0, CHUNK, _zrow, 0)
    for r0 in (0, 80, 160, 240, 320, 400, 480):
        pltpu.sync_copy(gbuf0, acc.at[pl.ds(s * RPT + r0, CHUNK)])
    pltpu.sync_copy(gbuf0.at[pl.ds(0, 64)], acc.at[pl.ds(s * RPT + 560, 64)])
    @pl.when(s == NS - 1)
    def _zero_tail():
        pltpu.sync_copy(gbuf0.at[pl.ds(0, 16)],
                        acc.at[pl.ds(s * RPT + 624, 16)])
    plsc.subcore_barrier()

    def _start_idx(k, b):
        @pl.when(k < NCHUNK)
        def _():
            pltpu.async_copy(cols_hbm.at[w, k], colv[b], semi[b])
            pltpu.async_copy(rows_hbm.at[w, k], rowv[b], semi[b])
            pltpu.async_copy(vals_hbm.at[w, k], valv[b], semi[b])

    def _wait_idx(b):
        pltpu.make_async_copy(cols_hbm.at[w, 0], colv[b], semi[b]).wait()
        pltpu.make_async_copy(rows_hbm.at[w, 0], rowv[b], semi[b]).wait()
        pltpu.make_async_copy(vals_hbm.at[w, 0], valv[b], semi[b]).wait()

    def _scale(b):
        for g in range(CHUNK // 16):
            v16 = valv[b][pl.ds(g * 16, 16)]
            for i in range(16):
                vb = _bcast_lane(v16, i)
                e = g * 16 + i
                for j in range(D // 16):
                    gbuf[b][e, pl.ds(j * 16, 16)] = (
                        gbuf[b][e, pl.ds(j * 16, 16)] * vb)

    def _wait_scatter(b):
        pltpu.make_async_copy(gbuf[b], acc.at[srow[b]], sems[b]).wait()

    def _step(k, b):
        bn = (b + 1) % 3
        bp = (b + 2) % 3
        _start_idx(k + 2, bp)
        @pl.when(k + 1 < NCHUNK)
        def _():
            _wait_idx(bn)
            @pl.when(k >= 2)
            def _():
                _wait_scatter(bn)
            pltpu.async_copy(m_hbm.at[colv[bn]], gbuf[bn], semg[bn])
        @pl.when(k < NCHUNK)
        def _():
            pltpu.make_async_copy(m_hbm.at[colv[b]], gbuf[b], semg[b]).wait()
            _scale(b)
            for i in range(CHUNK // 16):
                srow[b][pl.ds(i * 16, 16)] = rowv[b][pl.ds(i * 16, 16)]
            pltpu.async_copy(gbuf[b], acc.at[srow[b]], sems[b], add=True)
        @pl.when((k >= NCHUNK) & (k < NCHUNK + 3))
        def _():
            _wait_scatter(b)

    _start_idx(0, 0)
    _start_idx(1, 1)
    _wait_idx(0)
    pltpu.async_copy(m_hbm.at[colv[0]], gbuf[0], semg[0])

    def _tri(t, carry):
        k = 3 * t
        _step(k, 0)
        _step(k + 1, 1)
        _step(k + 2, 2)
        return carry
    lax.fori_loop(0, (NCHUNK + 2 + 2) // 3, _tri, 0)

    plsc.subcore_barrier()
    for r0 in (0, 80, 160, 240, 320, 400, 480):
        pltpu.sync_copy(acc.at[pl.ds(s * RPT + r0, CHUNK)], gbuf0)
        pltpu.sync_copy(gbuf0, out_hbm.at[c, pl.ds(s * RPT + r0, CHUNK)])
    pltpu.sync_copy(acc.at[pl.ds(s * RPT + 560, 64)], gbuf0.at[pl.ds(0, 64)])
    pltpu.sync_copy(gbuf0.at[pl.ds(0, 64)],
                    out_hbm.at[c, pl.ds(s * RPT + 560, 64)])
    @pl.when(s == NS - 1)
    def _drain_tail():
        pltpu.sync_copy(acc.at[pl.ds(s * RPT + 624, 16)],
                        gbuf0.at[pl.ds(0, 16)])
        pltpu.sync_copy(gbuf0.at[pl.ds(0, 16)],
                        out_hbm.at[c, pl.ds(s * RPT + 624, 16)])


@functools.lru_cache(maxsize=1)
def _get_spmm_sc():
    return pl.kernel(
        _spmm_body,
        out_type=jax.ShapeDtypeStruct((NC, N, D), jnp.float32),
        mesh=plsc.VectorSubcoreMesh(core_axis_name="c", subcore_axis_name="s",
                                    num_cores=NC, num_subcores=NS),
        scratch_types=(
            [pltpu.VMEM((CHUNK,), jnp.int32),
             pltpu.VMEM((CHUNK,), jnp.int32),
             pltpu.VMEM((CHUNK,), jnp.float32),
             pltpu.VMEM((CHUNK, D), jnp.float32),
             ] * 3
            + [pltpu.VMEM((CHUNK,), jnp.int32)] * 3
            + [pltpu.VMEM_SHARED((N, D), jnp.float32)]
            + [pltpu.SemaphoreType.DMA] * 9
        ),
    )


def _spmm_sc(m, cols3, rows3, vals2):
    return _get_spmm_sc()(m, cols3, rows3, vals2)


BR = 2000


def _mlp(x, w1, b1, w2, b2):
    h = jnp.maximum(jnp.dot(x, w1, preferred_element_type=jnp.float32) + b1,
                    0.0)
    return jnp.dot(h, w2, preferred_element_type=jnp.float32) + b2


def _stage_a_body(f2, f1, f0,
                  w12, b12, v12, c12,
                  w21, b21, v21, c21,
                  w11, b11, v11, c11,
                  w10, b10, v10, c10,
                  m1_o, t1_o, t0_o):
    x2 = _mlp(f2[...], w12[...], b12[...], v12[...], c12[...])
    m1_o[...] = _mlp(x2, w21[...], b21[...], v21[...], c21[...])
    t1_o[...] = _mlp(f1[...], w11[...], b11[...], v11[...], c11[...])
    t0_o[...] = _mlp(f0[...], w10[...], b10[...], v10[...], c10[...])


def _stage_b_body(t1, pa, pb, w20, b20, v20, c20, m0_o):
    x1 = t1[...] + pa[...] + pb[...]
    m0_o[...] = _mlp(x1, w20[...], b20[...], v20[...], c20[...])


def _stage_c_body(t0, pa, pb, out_o):
    out_o[...] = t0[...] + pa[...] + pb[...]


def _row_spec():
    return pl.BlockSpec((BR, D), lambda i: (i, 0))


def _w_spec():
    return pl.BlockSpec((D, D), lambda i: (0, 0))


def _b_spec():
    return pl.BlockSpec((1, D), lambda i: (0, 0))


_GRID = (N // BR,)

_stage_a = pl.pallas_call(
    _stage_a_body,
    grid=_GRID,
    in_specs=[_row_spec()] * 3 + [_w_spec(), _b_spec(), _w_spec(), _b_spec()] * 4,
    out_specs=[_row_spec()] * 3,
    out_shape=[jax.ShapeDtypeStruct((N, D), jnp.float32)] * 3,
)

_stage_b = pl.pallas_call(
    _stage_b_body,
    grid=_GRID,
    in_specs=[_row_spec()] * 3 + [_w_spec(), _b_spec(), _w_spec(), _b_spec()],
    out_specs=_row_spec(),
    out_shape=jax.ShapeDtypeStruct((N, D), jnp.float32),
)

_stage_c = pl.pallas_call(
    _stage_c_body,
    grid=_GRID,
    in_specs=[_row_spec()] * 3,
    out_specs=_row_spec(),
    out_shape=jax.ShapeDtypeStruct((N, D), jnp.float32),
)


def kernel(feat0, feat1, feat2, adj0_indices, adj0_values, adj1_indices,
           adj1_values, fc1_W1, fc1_b1, fc1_W2, fc1_b2, fc2_W1, fc2_b1,
           fc2_W2, fc2_b2):
    b = lambda x: x.reshape(1, D)
    cols1 = adj1_indices[1].reshape(NW, NCHUNK, CHUNK)
    rows1 = adj1_indices[0].reshape(NW, NCHUNK, CHUNK)
    vals1 = adj1_values.reshape(NW, NCHUNK, CHUNK)
    cols0 = adj0_indices[1].reshape(NW, NCHUNK, CHUNK)
    rows0 = adj0_indices[0].reshape(NW, NCHUNK, CHUNK)
    vals0 = adj0_values.reshape(NW, NCHUNK, CHUNK)

    m1, t1, t0 = _stage_a(
        feat2, feat1, feat0,
        fc1_W1[2], b(fc1_b1[2]), fc1_W2[2], b(fc1_b2[2]),
        fc2_W1[1], b(fc2_b1[1]), fc2_W2[1], b(fc2_b2[1]),
        fc1_W1[1], b(fc1_b1[1]), fc1_W2[1], b(fc1_b2[1]),
        fc1_W1[0], b(fc1_b1[0]), fc1_W2[0], b(fc1_b2[0]),
    )
    p1 = _spmm_sc(m1, cols1, rows1, vals1)
    m0 = _stage_b(t1, p1[0], p1[1],
                  fc2_W1[0], b(fc2_b1[0]), fc2_W2[0], b(fc2_b2[0]))
    p0 = _spmm_sc(m0, cols0, rows0, vals0)
    return _stage_c(t0, p0[0], p0[1])


# speedup vs baseline: 1.1043x; 1.1043x over previous
import functools

import jax
import jax.numpy as jnp
from jax import lax
from jax.experimental import pallas as pl
from jax.experimental.pallas import tpu as pltpu
from jax.experimental.pallas import tpu_sc as plsc

N = 10000
D = 128
E = 320000
NC = 2
NS = 16
NW = NC * NS
EPW = E // NW
CHUNK = 80
NCHUNK = EPW // CHUNK
RPT = 624


_GATHER_DNUMS = lax.GatherDimensionNumbers(
    offset_dims=(), collapsed_slice_dims=(0,), start_index_map=(0,))


def _bcast_lane(v16, i):
    idx = jnp.full((16, 1), i, jnp.int32)
    return lax.gather(v16, idx, _GATHER_DNUMS, (1,),
                      mode=lax.GatherScatterMode.PROMISE_IN_BOUNDS)


def _spmm_body(m_hbm, cols_hbm, rows_hbm, vals_hbm, out_hbm,
               colidx, rowv, valv, gbuf, rowv1, valv1, gbuf1, acc, sem, sem1):
    c = lax.axis_index("c")
    s = lax.axis_index("s")
    w = s * NC + c

    def _zrow(i, carry):
        for j in range(D // 16):
            gbuf[i, pl.ds(j * 16, 16)] = jnp.zeros((16,), jnp.float32)
        return carry
    lax.fori_loop(0, CHUNK, _zrow, 0)
    for r0 in (0, 80, 160, 240, 320, 400, 480):
        pltpu.sync_copy(gbuf, acc.at[pl.ds(s * RPT + r0, CHUNK)])
    pltpu.sync_copy(gbuf.at[pl.ds(0, 64)], acc.at[pl.ds(s * RPT + 560, 64)])
    @pl.when(s == NS - 1)
    def _zero_tail():
        pltpu.sync_copy(gbuf.at[pl.ds(0, 16)],
                        acc.at[pl.ds(s * RPT + 624, 16)])

    pltpu.sync_copy(cols_hbm.at[w], colidx)
    plsc.subcore_barrier()

    def _start(k, gb, rv, vv, sm):
        pltpu.async_copy(m_hbm.at[colidx.at[k]], gb, sm)
        pltpu.async_copy(rows_hbm.at[w, k], rv, sm)
        pltpu.async_copy(vals_hbm.at[w, k], vv, sm)

    def _wait(gb, rv, vv, sm):
        pltpu.make_async_copy(m_hbm.at[colidx.at[0]], gb, sm).wait()
        pltpu.make_async_copy(rows_hbm.at[w, 0], rv, sm).wait()
        pltpu.make_async_copy(vals_hbm.at[w, 0], vv, sm).wait()

    def _process(gb, rv, vv):
        for g in range(CHUNK // 16):
            v16 = vv[pl.ds(g * 16, 16)]
            for i in range(16):
                vb = _bcast_lane(v16, i)
                e = g * 16 + i
                for j in range(D // 16):
                    gb[e, pl.ds(j * 16, 16)] = gb[e, pl.ds(j * 16, 16)] * vb
        pltpu.sync_copy(gb, acc.at[rv], add=True)

    _start(0, gbuf, rowv, valv, sem)

    def _pair(t, carry):
        k0 = 2 * t
        _start(k0 + 1, gbuf1, rowv1, valv1, sem1)
        _wait(gbuf, rowv, valv, sem)
        _process(gbuf, rowv, valv)
        _start(k0 + 2, gbuf, rowv, valv, sem)
        _wait(gbuf1, rowv1, valv1, sem1)
        _process(gbuf1, rowv1, valv1)
        return carry
    lax.fori_loop(0, (NCHUNK - 1) // 2, _pair, 0)
    _wait(gbuf, rowv, valv, sem)
    _process(gbuf, rowv, valv)

    plsc.subcore_barrier()
    for r0 in (0, 80, 160, 240, 320, 400, 480):
        pltpu.sync_copy(acc.at[pl.ds(s * RPT + r0, CHUNK)], gbuf)
        pltpu.sync_copy(gbuf, out_hbm.at[c, pl.ds(s * RPT + r0, CHUNK)])
    pltpu.sync_copy(acc.at[pl.ds(s * RPT + 560, 64)], gbuf.at[pl.ds(0, 64)])
    pltpu.sync_copy(gbuf.at[pl.ds(0, 64)],
                    out_hbm.at[c, pl.ds(s * RPT + 560, 64)])
    @pl.when(s == NS - 1)
    def _drain_tail():
        pltpu.sync_copy(acc.at[pl.ds(s * RPT + 624, 16)],
                        gbuf.at[pl.ds(0, 16)])
        pltpu.sync_copy(gbuf.at[pl.ds(0, 16)],
                        out_hbm.at[c, pl.ds(s * RPT + 624, 16)])


@functools.lru_cache(maxsize=1)
def _get_spmm_sc():
    return pl.kernel(
        _spmm_body,
        out_type=jax.ShapeDtypeStruct((NC, N, D), jnp.float32),
        mesh=plsc.VectorSubcoreMesh(core_axis_name="c", subcore_axis_name="s",
                                    num_cores=NC, num_subcores=NS),
        scratch_types=[
            pltpu.VMEM((NCHUNK, CHUNK), jnp.int32),
            pltpu.VMEM((CHUNK,), jnp.int32),
            pltpu.VMEM((CHUNK,), jnp.float32),
            pltpu.VMEM((CHUNK, D), jnp.float32),
            pltpu.VMEM((CHUNK,), jnp.int32),
            pltpu.VMEM((CHUNK,), jnp.float32),
            pltpu.VMEM((CHUNK, D), jnp.float32),
            pltpu.VMEM_SHARED((N, D), jnp.float32),
            pltpu.SemaphoreType.DMA,
            pltpu.SemaphoreType.DMA,
        ],
    )


def _spmm_sc(m, cols3, rows3, vals2):
    return _get_spmm_sc()(m, cols3, rows3, vals2)


BR = 2000


def _mlp(x, w1, b1, w2, b2):
    h = jnp.maximum(jnp.dot(x, w1, preferred_element_type=jnp.float32) + b1,
                    0.0)
    return jnp.dot(h, w2, preferred_element_type=jnp.float32) + b2


def _mlp2_body(f, w1, b1, v1, c1, w2, b2, v2, c2, out_o):
    x = _mlp(f[...], w1[...], b1[...], v1[...], c1[...])
    out_o[...] = _mlp(x, w2[...], b2[...], v2[...], c2[...])


def _mlp1_body(f, w1, b1, v1, c1, out_o):
    out_o[...] = _mlp(f[...], w1[...], b1[...], v1[...], c1[...])


def _stage_b_body(t1, pa, pb, w20, b20, v20, c20, m0_o):
    x1 = t1[...] + pa[...] + pb[...]
    m0_o[...] = _mlp(x1, w20[...], b20[...], v20[...], c20[...])


def _stage_c_body(t0, pa, pb, out_o):
    out_o[...] = t0[...] + pa[...] + pb[...]


def _row_spec():
    return pl.BlockSpec((BR, D), lambda i: (i, 0))


def _w_spec():
    return pl.BlockSpec((D, D), lambda i: (0, 0))


def _b_spec():
    return pl.BlockSpec((1, D), lambda i: (0, 0))


_GRID = (N // BR,)
_WB = [_w_spec(), _b_spec(), _w_spec(), _b_spec()]

_mlp2_tc = pl.pallas_call(
    _mlp2_body,
    grid=_GRID,
    in_specs=[_row_spec()] + _WB * 2,
    out_specs=_row_spec(),
    out_shape=jax.ShapeDtypeStruct((N, D), jnp.float32),
)

_mlp1_tc = pl.pallas_call(
    _mlp1_body,
    grid=_GRID,
    in_specs=[_row_spec()] + _WB,
    out_specs=_row_spec(),
    out_shape=jax.ShapeDtypeStruct((N, D), jnp.float32),
)

_stage_b = pl.pallas_call(
    _stage_b_body,
    grid=_GRID,
    in_specs=[_row_spec()] * 3 + _WB,
    out_specs=_row_spec(),
    out_shape=jax.ShapeDtypeStruct((N, D), jnp.float32),
)

_stage_c = pl.pallas_call(
    _stage_c_body,
    grid=_GRID,
    in_specs=[_row_spec()] * 3,
    out_specs=_row_spec(),
    out_shape=jax.ShapeDtypeStruct((N, D), jnp.float32),
)


def kernel(feat0, feat1, feat2, adj0_indices, adj0_values, adj1_indices,
           adj1_values, fc1_W1, fc1_b1, fc1_W2, fc1_b2, fc2_W1, fc2_b1,
           fc2_W2, fc2_b2):
    b = lambda x: x.reshape(1, D)
    cols1 = adj1_indices[1].reshape(NW, NCHUNK, CHUNK)
    rows1 = adj1_indices[0].reshape(NW, NCHUNK, CHUNK)
    vals1 = adj1_values.reshape(NW, NCHUNK, CHUNK)
    cols0 = adj0_indices[1].reshape(NW, NCHUNK, CHUNK)
    rows0 = adj0_indices[0].reshape(NW, NCHUNK, CHUNK)
    vals0 = adj0_values.reshape(NW, NCHUNK, CHUNK)

    m1 = _mlp2_tc(feat2,
                  fc1_W1[2], b(fc1_b1[2]), fc1_W2[2], b(fc1_b2[2]),
                  fc2_W1[1], b(fc2_b1[1]), fc2_W2[1], b(fc2_b2[1]))
    p1 = _spmm_sc(m1, cols1, rows1, vals1)
    t1 = _mlp1_tc(feat1, fc1_W1[1], b(fc1_b1[1]), fc1_W2[1], b(fc1_b2[1]))
    t0 = _mlp1_tc(feat0, fc1_W1[0], b(fc1_b1[0]), fc1_W2[0], b(fc1_b2[0]))
    m0 = _stage_b(t1, p1[0], p1[1],
                  fc2_W1[0], b(fc2_b1[0]), fc2_W2[0], b(fc2_b2[0]))
    p0 = _spmm_sc(m0, cols0, rows0, vals0)
    return _stage_c(t0, p0[0], p0[1])
